# Initial kernel scaffold; baseline (speedup 1.0000x reference)
import functools

import jax
import jax.numpy as jnp
from jax import lax
from jax.experimental import pallas as pl
from jax.experimental.pallas import tpu as pltpu
from jax.experimental.pallas import tpu_sc as plsc

_NC = 2
_NS = 16
_NW = _NC * _NS

_CH = 128
_ROW_BLK = 1280

def _cdiv(a, b):
  return (a + b - 1) // b


def _make_sc_agg(n_pad, n_chunks, d):
  rows_per_tile = n_pad // _NS
  zero_chunks = rows_per_tile // 16
  out_chunks = rows_per_tile // _CH
  mesh = plsc.VectorSubcoreMesh(core_axis_name="c", subcore_axis_name="s")

  @functools.partial(
      pl.kernel,
      out_type=jax.ShapeDtypeStruct((_NC, n_pad, d), jnp.float32),
      mesh=mesh,
      scratch_types=[
          pltpu.VMEM((n_chunks, _CH), jnp.int32),
          pltpu.VMEM((n_chunks, _CH), jnp.int32),
          pltpu.VMEM((_CH, d), jnp.float32),
          pltpu.VMEM((16, d), jnp.float32),
          pltpu.VMEM_SHARED((n_pad, d), jnp.float32),
      ],
  )
  def sc_agg(h_hbm, src_hbm, dst_hbm, out_hbm, src_v, dst_v, rows_v, zbuf,
             agg_sh):
    cid = lax.axis_index("c")
    sid = lax.axis_index("s")
    wid = cid * _NS + sid

    pltpu.sync_copy(src_hbm.at[wid], src_v)
    pltpu.sync_copy(dst_hbm.at[wid], dst_v)

    zk = jnp.zeros((16,), jnp.float32)
    for r in range(16):
      for c in range(d // 16):
        zbuf[r, pl.ds(c * 16, 16)] = zk

    def zloop(r, carry):
      pltpu.sync_copy(zbuf, agg_sh.at[pl.ds((sid * zero_chunks + r) * 16, 16)])
      return carry

    lax.fori_loop(0, zero_chunks, zloop, 0)
    plsc.subcore_barrier()

    def eloop(j, carry):
      pltpu.sync_copy(h_hbm.at[src_v.at[j]], rows_v)
      pltpu.sync_copy(rows_v, agg_sh.at[dst_v.at[j]], add=True)
      return carry

    lax.fori_loop(0, n_chunks, eloop, 0)
    plsc.subcore_barrier()

    def oloop(k, carry):
      base = sid * rows_per_tile + k * _CH
      pltpu.sync_copy(agg_sh.at[pl.ds(base, _CH)], rows_v)
      pltpu.sync_copy(rows_v, out_hbm.at[cid].at[pl.ds(base, _CH)])
      return carry

    lax.fori_loop(0, out_chunks, oloop, 0)

  return sc_agg


def _dot_t(x, w):
  return lax.dot_general(x, w, (((1,), (1,)), ((), ())),
                         preferred_element_type=jnp.float32)


def _dense0_body(x_ref, w_ref, b_ref, o_ref):
  o_ref[...] = jnp.tanh(_dot_t(x_ref[...], w_ref[...]) + b_ref[...])


def _layer_body(agg_ref, h_ref, wrel_ref, wroot_ref, b_ref, o_ref):
  a = agg_ref[0] + agg_ref[1]
  o_ref[...] = jnp.tanh(_dot_t(a, wrel_ref[...]) +
                        _dot_t(h_ref[...], wroot_ref[...]) + b_ref[...])


def _final_body(h_ref, w_ref, b_ref, o_ref):
  o_ref[...] = jnp.maximum(
      _dot_t(h_ref[...], w_ref[...]) + b_ref[...], 0.0)


def _row_blocked(body, n_pad, d, in_specs):
  grid = (n_pad // _ROW_BLK,)
  return pl.pallas_call(
      body,
      grid=grid,
      in_specs=in_specs,
      out_specs=pl.BlockSpec((_ROW_BLK, d), lambda i: (i, 0)),
      out_shape=jax.ShapeDtypeStruct((n_pad, d), jnp.float32),
  )


def _mat_spec(d):
  return pl.BlockSpec((d, d), lambda i: (0, 0))


def _bias_spec(d):
  return pl.BlockSpec((1, d), lambda i: (0, 0))


def kernel(x, edge_index, W1, b1, Wrel, brel, Wroot, W2, b2):
  n, d = x.shape
  e = edge_index.shape[1]
  l = Wrel.shape[0]

  n_pad = _cdiv(n, _NS * _CH) * _NS * _CH
  n_chunks = _cdiv(e, _NW * _CH)
  e_pad = _NW * n_chunks * _CH

  x_pad = jnp.pad(x, ((0, n_pad - n), (0, 0)))
  src = jnp.concatenate(
      [edge_index[0], jnp.zeros((e_pad - e,), jnp.int32)]).reshape(
          _NW, n_chunks, _CH)
  dst = jnp.concatenate(
      [edge_index[1], jnp.full((e_pad - e,), n, jnp.int32)]).reshape(
          _NW, n_chunks, _CH)

  sc_agg = _make_sc_agg(n_pad, n_chunks, d)

  row_spec = pl.BlockSpec((_ROW_BLK, d), lambda i: (i, 0))
  agg_spec = pl.BlockSpec((_NC, _ROW_BLK, d), lambda i: (0, i, 0))

  dense0 = _row_blocked(_dense0_body, n_pad, d,
                        [row_spec, _mat_spec(d), _bias_spec(d)])
  layer = _row_blocked(_layer_body, n_pad, d,
                       [agg_spec, row_spec, _mat_spec(d), _mat_spec(d),
                        _bias_spec(d)])
  final = _row_blocked(_final_body, n_pad, d,
                       [row_spec, _mat_spec(d), _bias_spec(d)])

  h = dense0(x_pad, W1, b1.reshape(1, d))
  for i in range(l):
    agg = sc_agg(h, src, dst)
    h = layer(agg, h, Wrel[i], Wroot[i], brel[i].reshape(1, d))
  out = final(h, W2, b2.reshape(1, d))
  return out[:n]


# speedup vs baseline: 2.8601x; 2.8601x over previous
import functools

import jax
import jax.numpy as jnp
from jax import lax
from jax.experimental import pallas as pl
from jax.experimental.pallas import tpu as pltpu
from jax.experimental.pallas import tpu_sc as plsc

_NC = 2
_NS = 16
_NW = _NC * _NS

_CH = 128
_ROW_BLK = 1280

def _cdiv(a, b):
  return (a + b - 1) // b


def _make_sc_agg(n_pad, n_chunks, d):
  rows_per_tile = n_pad // _NS
  zero_chunks = rows_per_tile // 16
  out_chunks = rows_per_tile // _CH
  nk = d // 16
  mesh = plsc.VectorSubcoreMesh(core_axis_name="c", subcore_axis_name="s",
                                num_cores=_NC, num_subcores=_NS)

  @functools.partial(
      pl.kernel,
      out_type=jax.ShapeDtypeStruct((_NC, n_pad, d), jnp.float32),
      mesh=mesh,
      scratch_types=[
          pltpu.VMEM((n_chunks, _CH), jnp.int32),
          pltpu.VMEM((n_chunks, _CH), jnp.int32),
          pltpu.VMEM((2, _CH), jnp.float32),
          pltpu.VMEM((_CH, d), jnp.float32),
          pltpu.VMEM((16, d), jnp.float32),
          pltpu.VMEM_SHARED((n_pad, d), jnp.float32),
      ],
  )
  def sc_agg(h_hbm, src_hbm, dst_hbm, meta_hbm, out_hbm,
             src_v, dst_v, me_v, rows_v, zbuf, agg_sh):
    cid = lax.axis_index("c")
    sid = lax.axis_index("s")
    wid = cid * _NS + sid

    pltpu.sync_copy(src_hbm.at[wid], src_v)
    pltpu.sync_copy(dst_hbm.at[wid], dst_v)

    zk = jnp.zeros((16,), jnp.float32)
    for r in range(16):
      for c in range(nk):
        zbuf[r, pl.ds(c * 16, 16)] = zk

    def zloop(r, carry):
      pltpu.sync_copy(zbuf, agg_sh.at[pl.ds((sid * zero_chunks + r) * 16, 16)])
      return carry

    lax.fori_loop(0, zero_chunks, zloop, 0)
    plsc.subcore_barrier()

    def eloop(j, accs):
      pltpu.sync_copy(h_hbm.at[src_v.at[j]], rows_v)
      pltpu.sync_copy(meta_hbm.at[wid, j], me_v)

      def gloop(g, accs):
        sf = me_v[0, pl.ds(g * 16, 16)]
        ef = me_v[1, pl.ds(g * 16, 16)]
        for ei in range(16):
          s = sf[ei]
          f = ef[ei]
          row = g * 16 + ei
          accs = tuple(
              accs[k] * s + rows_v[row, pl.ds(k * 16, 16)]
              for k in range(nk))
          for k in range(nk):
            rows_v[row, pl.ds(k * 16, 16)] = accs[k] * f
        return accs

      accs = lax.fori_loop(0, _CH // 16, gloop, accs)
      pltpu.sync_copy(rows_v, agg_sh.at[dst_v.at[j]], add=True)
      return accs

    zero_accs = tuple(jnp.zeros((16,), jnp.float32) for _ in range(nk))
    lax.fori_loop(0, n_chunks, eloop, zero_accs)
    plsc.subcore_barrier()

    def oloop(k, carry):
      base = sid * rows_per_tile + k * _CH
      pltpu.sync_copy(agg_sh.at[pl.ds(base, _CH)], rows_v)
      pltpu.sync_copy(rows_v, out_hbm.at[cid].at[pl.ds(base, _CH)])
      return carry

    lax.fori_loop(0, out_chunks, oloop, 0)

  return sc_agg


def _dot_t(x, w):
  return lax.dot_general(x, w, (((1,), (1,)), ((), ())),
                         preferred_element_type=jnp.float32)


def _dense0_body(x_ref, w_ref, b_ref, o_ref):
  o_ref[...] = jnp.tanh(_dot_t(x_ref[...], w_ref[...]) + b_ref[...])


def _layer_body(agg_ref, h_ref, wrel_ref, wroot_ref, b_ref, o_ref):
  a = agg_ref[0] + agg_ref[1]
  o_ref[...] = jnp.tanh((_dot_t(a, wrel_ref[...]) + b_ref[...]) +
                        _dot_t(h_ref[...], wroot_ref[...]))


def _final_body(h_ref, w_ref, b_ref, o_ref):
  o_ref[...] = jnp.maximum(
      _dot_t(h_ref[...], w_ref[...]) + b_ref[...], 0.0)


def _row_blocked(body, n_pad, d, in_specs):
  grid = (n_pad // _ROW_BLK,)
  return pl.pallas_call(
      body,
      grid=grid,
      in_specs=in_specs,
      out_specs=pl.BlockSpec((_ROW_BLK, d), lambda i: (i, 0)),
      out_shape=jax.ShapeDtypeStruct((n_pad, d), jnp.float32),
  )


def _mat_spec(d):
  return pl.BlockSpec((d, d), lambda i: (0, 0))


def _bias_spec(d):
  return pl.BlockSpec((1, d), lambda i: (0, 0))


def kernel(x, edge_index, W1, b1, Wrel, brel, Wroot, W2, b2):
  n, d = x.shape
  e = edge_index.shape[1]
  l = Wrel.shape[0]

  n_pad = _cdiv(n, _NS * _CH) * _NS * _CH
  n_chunks = _cdiv(e, _NW * _CH)
  e_pad = _NW * n_chunks * _CH
  t_edges = n_chunks * _CH

  x_pad = jnp.pad(x, ((0, n_pad - n), (0, 0)))

  order = jnp.argsort(edge_index[1], stable=True)
  ds = jnp.concatenate(
      [edge_index[1][order], jnp.full((e_pad - e,), n, jnp.int32)])
  ss = jnp.concatenate(
      [edge_index[0][order], jnp.zeros((e_pad - e,), jnp.int32)])
  pos = jnp.arange(e_pad, dtype=jnp.int32)
  prev = jnp.concatenate([jnp.array([-1], jnp.int32), ds[:-1]])
  nxt = jnp.concatenate([ds[1:], jnp.array([-1], jnp.int32)])
  in_tile_pos = pos % t_edges
  samef = ((ds == prev) & (in_tile_pos != 0)).astype(jnp.float32)
  endf = ((ds != nxt) | (in_tile_pos == t_edges - 1)).astype(jnp.float32)

  src3 = ss.reshape(_NW, n_chunks, _CH)
  dst3 = ds.reshape(_NW, n_chunks, _CH)
  meta4 = jnp.stack(
      [samef.reshape(_NW, n_chunks, _CH), endf.reshape(_NW, n_chunks, _CH)],
      axis=2)

  sc_agg = _make_sc_agg(n_pad, n_chunks, d)

  row_spec = pl.BlockSpec((_ROW_BLK, d), lambda i: (i, 0))
  agg_spec = pl.BlockSpec((_NC, _ROW_BLK, d), lambda i: (0, i, 0))

  dense0 = _row_blocked(_dense0_body, n_pad, d,
                        [row_spec, _mat_spec(d), _bias_spec(d)])
  layer = _row_blocked(_layer_body, n_pad, d,
                       [agg_spec, row_spec, _mat_spec(d), _mat_spec(d),
                        _bias_spec(d)])
  final = _row_blocked(_final_body, n_pad, d,
                       [row_spec, _mat_spec(d), _bias_spec(d)])

  h = dense0(x_pad, W1, b1.reshape(1, d))
  for i in range(l):
    agg = sc_agg(h, src3, dst3, meta4)
    h = layer(agg, h, Wrel[i], Wroot[i], brel[i].reshape(1, d))
  out = final(h, W2, b2.reshape(1, d))
  return out[:n]
